# baseline (speedup 1.0000x reference)
import functools

import jax
import jax.numpy as jnp
from jax import lax
from jax.experimental import pallas as pl
from jax.experimental.pallas import tpu as pltpu
from jax.experimental.pallas import tpu_sc as plsc

_N = 10000
_NP = 10240
_E = 320000
_D = 128
_NC = 2
_NS = 16
_NW = _NC * _NS
_CL = 128
_NB = 4
_DH = _D // _NC
_CH = -(-_E // (_NS * _CL * _NB)) * _NB
_EP = _NS * _CH * _CL
_RT = _NP // _NS

def _deg_body(dst_hbm, out_hbm, dstv, ones_v, zb, deg_sh):
    c = lax.axis_index("c")
    s = lax.axis_index("s")

    def _z(i, carry):
        zb[pl.ds(i * 16, 16)] = jnp.zeros((16,), jnp.float32)
        return carry

    lax.fori_loop(0, _RT // 16, _z, 0)

    def _o(i, carry):
        ones_v[pl.ds(i * 16, 16)] = jnp.ones((16,), jnp.float32)
        return carry

    lax.fori_loop(0, _CL // 16, _o, 0)

    pltpu.sync_copy(zb, deg_sh.at[pl.ds(s * _RT, _RT)])
    plsc.subcore_barrier()

    pltpu.sync_copy(dst_hbm.at[s], dstv)

    def _scat(j, carry):
        pltpu.sync_copy(ones_v, deg_sh.at[dstv.at[j]], add=True)
        return carry

    lax.fori_loop(0, _CH, _scat, 0)
    plsc.subcore_barrier()

    @pl.when(c == 0)
    def _():
        pltpu.sync_copy(deg_sh.at[pl.ds(s * _RT, _RT)],
                        out_hbm.at[pl.ds(s * _RT, _RT)])


def _scatter_body(g_hbm, src_hbm, dst_hbm, out_hbm, srcv, dstv, rows, agg_sh,
                  gsem, ssem):
    c = lax.axis_index("c")
    s = lax.axis_index("s")

    zbuf = rows.at[0]

    def _z(i, carry):
        zbuf[i // (_DH // 16), pl.ds((i % (_DH // 16)) * 16, 16)] = (
            jnp.zeros((16,), jnp.float32))
        return carry

    lax.fori_loop(0, _CL * (_DH // 16), _z, 0)
    for k in range(_RT // _CL):
        pltpu.sync_copy(zbuf, agg_sh.at[pl.ds(s * _RT + k * _CL, _CL)])
    plsc.subcore_barrier()

    pltpu.sync_copy(src_hbm.at[c, s], srcv)
    pltpu.sync_copy(dst_hbm.at[s], dstv)

    for b in range(_NB):
        pltpu.async_copy(g_hbm.at[srcv.at[b]], rows.at[b], gsem.at[b])

    def _round(t, carry):
        for b in range(_NB):
            j = t * _NB + b
            pltpu.make_async_copy(g_hbm.at[srcv.at[j]], rows.at[b],
                                  gsem.at[b]).wait()
            pltpu.async_copy(rows.at[b], agg_sh.at[dstv.at[j]], ssem.at[b],
                             add=True)
        for b in range(_NB):
            j = t * _NB + b
            pltpu.make_async_copy(rows.at[b], agg_sh.at[dstv.at[j]],
                                  ssem.at[b]).wait()

            @pl.when(t < _CH // _NB - 1)
            def _():
                pltpu.async_copy(g_hbm.at[srcv.at[j + _NB]], rows.at[b],
                                 gsem.at[b])

        return carry

    lax.fori_loop(0, _CH // _NB, _round, 0)
    plsc.subcore_barrier()
    for k in range(_RT // _CL):
        pltpu.sync_copy(agg_sh.at[pl.ds(s * _RT + k * _CL, _CL)],
                        out_hbm.at[c, pl.ds(s * _RT + k * _CL, _CL)])


@functools.cache
def _sc_kernels():
    mesh = plsc.VectorSubcoreMesh(core_axis_name="c", subcore_axis_name="s",
                                  num_cores=_NC, num_subcores=_NS)
    deg_kernel = pl.kernel(
        _deg_body,
        out_type=jax.ShapeDtypeStruct((_NP,), jnp.float32),
        mesh=mesh,
        scratch_types=[
            pltpu.VMEM((_CH, _CL), jnp.int32),
            pltpu.VMEM((_CL,), jnp.float32),
            pltpu.VMEM((_RT,), jnp.float32),
            pltpu.VMEM_SHARED((_NP,), jnp.float32),
        ],
    )
    scatter_kernel = pl.kernel(
        _scatter_body,
        out_type=jax.ShapeDtypeStruct((_NC, _NP, _DH), jnp.float32),
        mesh=mesh,
        scratch_types=[
            pltpu.VMEM((_CH, _CL), jnp.int32),
            pltpu.VMEM((_CH, _CL), jnp.int32),
            pltpu.VMEM((_NB, _CL, _DH), jnp.float32),
            pltpu.VMEM_SHARED((_NP, _DH), jnp.float32),
            pltpu.SemaphoreType.DMA((_NB,)),
            pltpu.SemaphoreType.DMA((_NB,)),
        ],
        compiler_params=pltpu.CompilerParams(use_tc_tiling_on_sc=False),
    )
    return deg_kernel, scatter_kernel


def _lrelu(v):
    return jnp.where(v >= 0, v, 0.01 * v)


_R = 1024


def _mlp_body(x_ref, w1_ref, b1_ref, w2_ref, b2_ref, dp_ref,
              h_ref, g_ref, dinv_ref):
    a = _lrelu(jnp.dot(x_ref[...], w1_ref[...],
                       preferred_element_type=jnp.float32) + b1_ref[...])
    h = _lrelu(jnp.dot(a, w2_ref[...],
                       preferred_element_type=jnp.float32) + b2_ref[...])
    deg = jnp.maximum(dp_ref[...], 1.0)
    dinv = lax.rsqrt(deg)
    h_ref[...] = h
    g_ref[...] = h * dinv
    dinv_ref[...] = dinv


_mlp_kernel = pl.pallas_call(
    _mlp_body,
    grid=(_NP // _R,),
    in_specs=[
        pl.BlockSpec((_R, _D), lambda i: (i, 0)),
        pl.BlockSpec((_D, _D), lambda i: (0, 0)),
        pl.BlockSpec((1, _D), lambda i: (0, 0)),
        pl.BlockSpec((_D, _D), lambda i: (0, 0)),
        pl.BlockSpec((1, _D), lambda i: (0, 0)),
        pl.BlockSpec((_R, 1), lambda i: (i, 0)),
    ],
    out_specs=[
        pl.BlockSpec((_R, _D), lambda i: (i, 0)),
        pl.BlockSpec((_R, _D), lambda i: (i, 0)),
        pl.BlockSpec((_R, 1), lambda i: (i, 0)),
    ],
    out_shape=[
        jax.ShapeDtypeStruct((_NP, _D), jnp.float32),
        jax.ShapeDtypeStruct((_NP, _D), jnp.float32),
        jax.ShapeDtypeStruct((_NP, 1), jnp.float32),
    ],
)


def _lap_body(h_ref, agg_ref, dinv_ref, f_ref, g_ref):
    dinv = dinv_ref[...]
    agg = jnp.concatenate([agg_ref[0], agg_ref[1]], axis=-1)
    f = h_ref[...] - agg * dinv
    f_ref[...] = f
    g_ref[...] = f * dinv


_lap_kernel = pl.pallas_call(
    _lap_body,
    grid=(_NP // _R,),
    in_specs=[
        pl.BlockSpec((_R, _D), lambda i: (i, 0)),
        pl.BlockSpec((_NC, _R, _DH), lambda i: (0, i, 0)),
        pl.BlockSpec((_R, 1), lambda i: (i, 0)),
    ],
    out_specs=[
        pl.BlockSpec((_R, _D), lambda i: (i, 0)),
        pl.BlockSpec((_R, _D), lambda i: (i, 0)),
    ],
    out_shape=[
        jax.ShapeDtypeStruct((_NP, _D), jnp.float32),
        jax.ShapeDtypeStruct((_NP, _D), jnp.float32),
    ],
)


def _out_body(h_ref, f1_ref, agg_ref, dinv_ref, w3_ref, b3_ref, o_ref):
    f1 = f1_ref[...]
    agg = jnp.concatenate([agg_ref[0], agg_ref[1]], axis=-1)
    f2 = f1 - agg * dinv_ref[...]
    h = h_ref[...]
    acc0 = 3.0 * h - 3.0 * f1 + 0.75 * f2
    acc1 = 3.0 * f1 - 1.5 * f2
    acc2 = 0.75 * f2
    o = (jnp.dot(acc0, w3_ref[0], preferred_element_type=jnp.float32)
         + jnp.dot(acc1, w3_ref[1], preferred_element_type=jnp.float32)
         + jnp.dot(acc2, w3_ref[2], preferred_element_type=jnp.float32)
         + b3_ref[...])
    o_ref[...] = _lrelu(o)


_out_kernel = pl.pallas_call(
    _out_body,
    grid=(_NP // _R,),
    in_specs=[
        pl.BlockSpec((_R, _D), lambda i: (i, 0)),
        pl.BlockSpec((_R, _D), lambda i: (i, 0)),
        pl.BlockSpec((_NC, _R, _DH), lambda i: (0, i, 0)),
        pl.BlockSpec((_R, 1), lambda i: (i, 0)),
        pl.BlockSpec((3, _D, _D), lambda i: (0, 0, 0)),
        pl.BlockSpec((1, _D), lambda i: (0, 0)),
    ],
    out_specs=pl.BlockSpec((_R, _D), lambda i: (i, 0)),
    out_shape=jax.ShapeDtypeStruct((_NP, _D), jnp.float32),
)


def kernel(x, edge_index, W1, b1, W2, b2, W3, b3):
    src = edge_index[0]
    dst = edge_index[1]
    pad = _EP - _E
    padv = jnp.full((pad,), _N, jnp.int32)
    srcp = jnp.concatenate([src, padv])
    dstp = jnp.concatenate([dst, padv]).reshape(_NS, _CH, _CL)
    srcx = jnp.stack([2 * srcp, 2 * srcp + 1]).reshape(_NC, _NS, _CH, _CL)
    xp = jnp.pad(x, ((0, _NP - _N), (0, 0)))

    _deg_kernel, _scatter_kernel = _sc_kernels()
    deg = _deg_kernel(dstp)
    h, g, dinv = _mlp_kernel(xp, W1, b1.reshape(1, _D), W2, b2.reshape(1, _D),
                             deg[:, None])
    agg1 = _scatter_kernel(g.reshape(_NC * _NP, _DH), srcx, dstp)
    f1, g1 = _lap_kernel(h, agg1, dinv)
    agg2 = _scatter_kernel(g1.reshape(_NC * _NP, _DH), srcx, dstp)
    out = _out_kernel(h, f1, agg2, dinv, W3.reshape(3, _D, _D),
                      b3.reshape(1, _D))
    return out[:_N]


# speedup vs baseline: 1.2650x; 1.2650x over previous
import functools

import jax
import jax.numpy as jnp
from jax import lax
from jax.experimental import pallas as pl
from jax.experimental.pallas import tpu as pltpu
from jax.experimental.pallas import tpu_sc as plsc

_N = 10000
_NP = 10240
_E = 320000
_D = 128
_NC = 2
_NS = 16
_NW = _NC * _NS
_CL = 64
_NB = 2
_CH = -(-_E // (_NW * _CL * _NB)) * _NB
_EP = _NW * _CH * _CL
_RT = _NP // _NS

def _deg_body(dst_hbm, out_hbm, dstv, ones_v, zb, deg_sh):
    c = lax.axis_index("c")
    s = lax.axis_index("s")
    wid = s * _NC + c

    def _z(i, carry):
        zb[pl.ds(i * 16, 16)] = jnp.zeros((16,), jnp.float32)
        return carry

    lax.fori_loop(0, _RT // 16, _z, 0)

    def _o(i, carry):
        ones_v[pl.ds(i * 16, 16)] = jnp.ones((16,), jnp.float32)
        return carry

    lax.fori_loop(0, _CL // 16, _o, 0)

    pltpu.sync_copy(zb, deg_sh.at[pl.ds(s * _RT, _RT)])
    plsc.subcore_barrier()

    pltpu.sync_copy(dst_hbm.at[wid], dstv)

    def _scat(j, carry):
        pltpu.sync_copy(ones_v, deg_sh.at[dstv.at[j]], add=True)
        return carry

    lax.fori_loop(0, _CH, _scat, 0)
    plsc.subcore_barrier()
    pltpu.sync_copy(deg_sh.at[pl.ds(s * _RT, _RT)],
                    out_hbm.at[c, pl.ds(s * _RT, _RT)])


def _scatter_body(g_hbm, src_hbm, dst_hbm, out_hbm, srcv, dstv, rows, agg_sh,
                  gsem, ssem):
    c = lax.axis_index("c")
    s = lax.axis_index("s")

    wid = s * _NC + c

    zbuf = rows.at[0]

    def _z(i, carry):
        zbuf[i // (_D // 16), pl.ds((i % (_D // 16)) * 16, 16)] = (
            jnp.zeros((16,), jnp.float32))
        return carry

    lax.fori_loop(0, _CL * (_D // 16), _z, 0)
    for k in range(_RT // _CL):
        pltpu.sync_copy(zbuf, agg_sh.at[pl.ds(s * _RT + k * _CL, _CL)])
    plsc.subcore_barrier()

    pltpu.sync_copy(src_hbm.at[wid], srcv)
    pltpu.sync_copy(dst_hbm.at[wid], dstv)

    for b in range(_NB):
        pltpu.async_copy(g_hbm.at[srcv.at[b]], rows.at[b], gsem.at[b])

    def _round(t, carry):
        for b in range(_NB):
            j = t * _NB + b
            pltpu.make_async_copy(g_hbm.at[srcv.at[j]], rows.at[b],
                                  gsem.at[b]).wait()
            pltpu.async_copy(rows.at[b], agg_sh.at[dstv.at[j]], ssem.at[b],
                             add=True)
        for b in range(_NB):
            j = t * _NB + b
            pltpu.make_async_copy(rows.at[b], agg_sh.at[dstv.at[j]],
                                  ssem.at[b]).wait()

            @pl.when(t < _CH // _NB - 1)
            def _():
                pltpu.async_copy(g_hbm.at[srcv.at[j + _NB]], rows.at[b],
                                 gsem.at[b])

        return carry

    lax.fori_loop(0, _CH // _NB, _round, 0)
    plsc.subcore_barrier()
    for k in range(_RT // _CL):
        pltpu.sync_copy(agg_sh.at[pl.ds(s * _RT + k * _CL, _CL)],
                        out_hbm.at[c, pl.ds(s * _RT + k * _CL, _CL)])


@functools.cache
def _sc_kernels():
    mesh = plsc.VectorSubcoreMesh(core_axis_name="c", subcore_axis_name="s",
                                  num_cores=_NC, num_subcores=_NS)
    deg_kernel = pl.kernel(
        _deg_body,
        out_type=jax.ShapeDtypeStruct((_NC, _NP), jnp.float32),
        mesh=mesh,
        scratch_types=[
            pltpu.VMEM((_CH, _CL), jnp.int32),
            pltpu.VMEM((_CL,), jnp.float32),
            pltpu.VMEM((_RT,), jnp.float32),
            pltpu.VMEM_SHARED((_NP,), jnp.float32),
        ],
    )
    scatter_kernel = pl.kernel(
        _scatter_body,
        out_type=jax.ShapeDtypeStruct((_NC, _NP, _D), jnp.float32),
        mesh=mesh,
        scratch_types=[
            pltpu.VMEM((_CH, _CL), jnp.int32),
            pltpu.VMEM((_CH, _CL), jnp.int32),
            pltpu.VMEM((_NB, _CL, _D), jnp.float32),
            pltpu.VMEM_SHARED((_NP, _D), jnp.float32),
            pltpu.SemaphoreType.DMA((_NB,)),
            pltpu.SemaphoreType.DMA((_NB,)),
        ],
        compiler_params=pltpu.CompilerParams(use_tc_tiling_on_sc=False),
    )
    return deg_kernel, scatter_kernel


def _lrelu(v):
    return jnp.where(v >= 0, v, 0.01 * v)


_R = 1024


def _mlp_body(x_ref, w1_ref, b1_ref, w2_ref, b2_ref, dp_ref,
              h_ref, g_ref, dinv_ref):
    a = _lrelu(jnp.dot(x_ref[...], w1_ref[...],
                       preferred_element_type=jnp.float32) + b1_ref[...])
    h = _lrelu(jnp.dot(a, w2_ref[...],
                       preferred_element_type=jnp.float32) + b2_ref[...])
    deg = jnp.maximum(dp_ref[0] + dp_ref[1], 1.0)
    dinv = lax.rsqrt(deg)
    h_ref[...] = h
    g_ref[...] = h * dinv
    dinv_ref[...] = dinv


_mlp_kernel = pl.pallas_call(
    _mlp_body,
    grid=(_NP // _R,),
    in_specs=[
        pl.BlockSpec((_R, _D), lambda i: (i, 0)),
        pl.BlockSpec((_D, _D), lambda i: (0, 0)),
        pl.BlockSpec((1, _D), lambda i: (0, 0)),
        pl.BlockSpec((_D, _D), lambda i: (0, 0)),
        pl.BlockSpec((1, _D), lambda i: (0, 0)),
        pl.BlockSpec((_NC, _R, 1), lambda i: (0, i, 0)),
    ],
    out_specs=[
        pl.BlockSpec((_R, _D), lambda i: (i, 0)),
        pl.BlockSpec((_R, _D), lambda i: (i, 0)),
        pl.BlockSpec((_R, 1), lambda i: (i, 0)),
    ],
    out_shape=[
        jax.ShapeDtypeStruct((_NP, _D), jnp.float32),
        jax.ShapeDtypeStruct((_NP, _D), jnp.float32),
        jax.ShapeDtypeStruct((_NP, 1), jnp.float32),
    ],
)


def _lap_body(h_ref, agg_ref, dinv_ref, f_ref, g_ref):
    dinv = dinv_ref[...]
    f = h_ref[...] - (agg_ref[0] + agg_ref[1]) * dinv
    f_ref[...] = f
    g_ref[...] = f * dinv


_lap_kernel = pl.pallas_call(
    _lap_body,
    grid=(_NP // _R,),
    in_specs=[
        pl.BlockSpec((_R, _D), lambda i: (i, 0)),
        pl.BlockSpec((_NC, _R, _D), lambda i: (0, i, 0)),
        pl.BlockSpec((_R, 1), lambda i: (i, 0)),
    ],
    out_specs=[
        pl.BlockSpec((_R, _D), lambda i: (i, 0)),
        pl.BlockSpec((_R, _D), lambda i: (i, 0)),
    ],
    out_shape=[
        jax.ShapeDtypeStruct((_NP, _D), jnp.float32),
        jax.ShapeDtypeStruct((_NP, _D), jnp.float32),
    ],
)


def _out_body(h_ref, f1_ref, agg_ref, dinv_ref, w3_ref, b3_ref, o_ref):
    f1 = f1_ref[...]
    f2 = f1 - (agg_ref[0] + agg_ref[1]) * dinv_ref[...]
    h = h_ref[...]
    acc0 = 3.0 * h - 3.0 * f1 + 0.75 * f2
    acc1 = 3.0 * f1 - 1.5 * f2
    acc2 = 0.75 * f2
    o = (jnp.dot(acc0, w3_ref[0], preferred_element_type=jnp.float32)
         + jnp.dot(acc1, w3_ref[1], preferred_element_type=jnp.float32)
         + jnp.dot(acc2, w3_ref[2], preferred_element_type=jnp.float32)
         + b3_ref[...])
    o_ref[...] = _lrelu(o)


_out_kernel = pl.pallas_call(
    _out_body,
    grid=(_NP // _R,),
    in_specs=[
        pl.BlockSpec((_R, _D), lambda i: (i, 0)),
        pl.BlockSpec((_R, _D), lambda i: (i, 0)),
        pl.BlockSpec((_NC, _R, _D), lambda i: (0, i, 0)),
        pl.BlockSpec((_R, 1), lambda i: (i, 0)),
        pl.BlockSpec((3, _D, _D), lambda i: (0, 0, 0)),
        pl.BlockSpec((1, _D), lambda i: (0, 0)),
    ],
    out_specs=pl.BlockSpec((_R, _D), lambda i: (i, 0)),
    out_shape=jax.ShapeDtypeStruct((_NP, _D), jnp.float32),
)


def kernel(x, edge_index, W1, b1, W2, b2, W3, b3):
    src = edge_index[0]
    dst = edge_index[1]
    pad = _EP - _E
    padv = jnp.full((pad,), _N, jnp.int32)
    srcp = jnp.concatenate([src, padv]).reshape(_NW, _CH, _CL)
    dstp = jnp.concatenate([dst, padv]).reshape(_NW, _CH, _CL)
    xp = jnp.pad(x, ((0, _NP - _N), (0, 0)))

    _deg_kernel, _scatter_kernel = _sc_kernels()
    deg_parts = _deg_kernel(dstp)
    h, g, dinv = _mlp_kernel(xp, W1, b1.reshape(1, _D), W2, b2.reshape(1, _D),
                             deg_parts[..., None])
    agg1 = _scatter_kernel(g, srcp, dstp)
    f1, g1 = _lap_kernel(h, agg1, dinv)
    agg2 = _scatter_kernel(g1, srcp, dstp)
    out = _out_kernel(h, f1, agg2, dinv, W3.reshape(3, _D, _D),
                      b3.reshape(1, _D))
    return out[:_N]
